# baseline (speedup 1.0000x reference)
import functools

import jax
import jax.numpy as jnp
from jax import lax
from jax.experimental import pallas as pl
from jax.experimental.pallas import tpu as pltpu
from jax.experimental.pallas import tpu_sc as plsc

N = 10000
E = 320000
D = 128
H = 128
C = 32
G = 64

NC = 2
NS = 16
NW = NC * NS
BLK = 128
NBLK = 80
EPT = NBLK * BLK
EPAD = NW * EPT
NPAD = 10240
RPT = NPAD // NS

_mesh = plsc.VectorSubcoreMesh(core_axis_name="c", subcore_axis_name="s")


def _segment_sum_sc(h, e4, zblk):
    @functools.partial(
        pl.kernel,
        out_type=jax.ShapeDtypeStruct((NC * NPAD, H), jnp.float32),
        mesh=_mesh,
        scratch_types=[
            pltpu.VMEM((2, 4, 2, BLK), jnp.int32),
            pltpu.VMEM((BLK, H), jnp.float32),
            pltpu.VMEM((BLK, H), jnp.float32),
            pltpu.SemaphoreType.DMA,
            pltpu.SemaphoreType.DMA,
            pltpu.SemaphoreType.DMA,
            pltpu.SemaphoreType.DMA,
            pltpu.VMEM_SHARED((NPAD, H), jnp.float32),
        ],
    )
    def k(h_hbm, e_hbm, z_hbm, out_hbm, ebuf, rows0, rows1,
          si0, si1, sg0, sg1, acc):
        c = lax.axis_index("c")
        s = lax.axis_index("s")
        tid = c * NS + s
        rows = (rows0, rows1)
        semi = (si0, si1)
        semg = (sg0, sg1)
        pltpu.sync_copy(z_hbm, acc.at[pl.ds(s * RPT, RPT)])
        pltpu.async_copy(e_hbm.at[tid, 0], ebuf.at[0], semi[0])
        pltpu.async_copy(e_hbm.at[tid, 1], ebuf.at[1], semi[1])
        plsc.subcore_barrier()
        pltpu.make_async_copy(e_hbm.at[tid, 0], ebuf.at[0], semi[0]).wait()
        pltpu.async_copy(h_hbm.at[ebuf.at[0, 0, 0]], rows0, semg[0])

        def phase(j, p, pref_idx, pref_gather):
            half, slot = p // 4, p % 4
            b = p % 2
            if pref_gather:
                h1, s1 = ((p + 1) % 8) // 4, (p + 1) % 4
                if s1 == 0:
                    pltpu.make_async_copy(e_hbm.at[tid, (j + 1) // 4],
                                          ebuf.at[h1], semi[h1]).wait()
                pltpu.async_copy(h_hbm.at[ebuf.at[h1, s1, 0]], rows[1 - b],
                                 semg[1 - b])
            pltpu.make_async_copy(h_hbm.at[ebuf.at[half, slot, 0]], rows[b],
                                  semg[b]).wait()
            pltpu.sync_copy(rows[b], acc.at[ebuf.at[half, slot, 1]], add=True)
            if pref_idx and slot == 3:
                pltpu.async_copy(e_hbm.at[tid, (j + 5) // 4], ebuf.at[half],
                                 semi[half])

        @pl.loop(0, NBLK - 8, step=8)
        def _(j):
            for p in range(8):
                phase(j + p, p, True, True)

        for q in range(8):
            phase(NBLK - 8 + q, q, False, q < 7)

        plsc.subcore_barrier()
        pltpu.sync_copy(acc.at[pl.ds(s * RPT, RPT)],
                        out_hbm.at[pl.ds(c * NPAD + s * RPT, RPT)])

    return k(h, e4, zblk)


def _dense_body(p_ref, h_ref, wr_ref, br_ref, wt_ref, g_ref, b_ref, o_ref):
    agg = p_ref[0, :N] + p_ref[1, :N]
    y = (lax.dot(agg, wr_ref[...], precision=lax.Precision.HIGHEST,
                 preferred_element_type=jnp.float32)
         + lax.dot(h_ref[...], wt_ref[...], precision=lax.Precision.HIGHEST,
                   preferred_element_type=jnp.float32)
         + br_ref[...])
    y = jnp.maximum(y, 0.0)
    mean = jnp.mean(y, axis=0, keepdims=True)
    var = jnp.mean((y - mean) ** 2, axis=0, keepdims=True)
    o_ref[...] = (y - mean) * lax.rsqrt(var + 1e-5) * g_ref[...] + b_ref[...]


def _dense_tc(parts, h, wr, br, wt, gamma, beta):
    parts = parts.reshape(NC, NPAD, H)
    return pl.pallas_call(
        _dense_body,
        out_shape=jax.ShapeDtypeStruct((N, H), jnp.float32),
    )(parts, h, wr, br, wt, gamma, beta)


def _dense_pool_body(p_ref, h_ref, wr_ref, br_ref, wt_ref, g_ref, b_ref,
                     batch_ref, wl_ref, bl_ref, o_ref):
    agg = p_ref[0, :N] + p_ref[1, :N]
    y = (lax.dot(agg, wr_ref[...], precision=lax.Precision.HIGHEST,
                 preferred_element_type=jnp.float32)
         + lax.dot(h_ref[...], wt_ref[...], precision=lax.Precision.HIGHEST,
                   preferred_element_type=jnp.float32)
         + br_ref[...])
    y = jnp.maximum(y, 0.0)
    mean = jnp.mean(y, axis=0, keepdims=True)
    var = jnp.mean((y - mean) ** 2, axis=0, keepdims=True)
    y = (y - mean) * lax.rsqrt(var + 1e-5) * g_ref[...] + b_ref[...]
    onehot = (batch_ref[...] ==
              lax.broadcasted_iota(jnp.int32, (G, N), 0)).astype(jnp.float32)
    sums = lax.dot(onehot, y, precision=lax.Precision.HIGHEST,
                   preferred_element_type=jnp.float32)
    counts = jnp.sum(onehot, axis=1, keepdims=True)
    pooled = sums / jnp.maximum(counts, 1.0)
    o_ref[...] = (lax.dot(pooled, wl_ref[...], precision=lax.Precision.HIGHEST,
                          preferred_element_type=jnp.float32) + bl_ref[...])


def _dense_pool_tc(parts, h, wr, br, wt, gamma, beta, batch2, wl, bl):
    parts = parts.reshape(NC, NPAD, H)
    return pl.pallas_call(
        _dense_pool_body,
        out_shape=jax.ShapeDtypeStruct((G, C), jnp.float32),
    )(parts, h, wr, br, wt, gamma, beta, batch2, wl, bl)


def kernel(x, edge_index, batch, W_rel1, b_rel1, W_root1, W_rel2, b_rel2,
           W_root2, W_rel3, b_rel3, W_root3, bn_gamma, bn_beta, W_lin, b_lin):
    ppt = EPT - E // NW
    padsrc = jnp.broadcast_to((jnp.arange(ppt, dtype=jnp.int32) * 41) % N,
                              (NW, ppt))
    srcp = jnp.concatenate(
        [edge_index[0].reshape(NW, E // NW), padsrc], axis=1)
    drpt = (NPAD - N) // NS
    padrow = (N + (jnp.arange(NW, dtype=jnp.int32) % NS)[:, None] * drpt
              + (jnp.arange(ppt, dtype=jnp.int32) % drpt)[None, :])
    dstp = jnp.concatenate(
        [edge_index[1].reshape(NW, E // NW), padrow], axis=1)
    e4 = jnp.stack([srcp.reshape(NW, NBLK, BLK),
                    dstp.reshape(NW, NBLK, BLK)],
                   axis=2).reshape(NW, NBLK // 4, 4, 2, BLK)
    zblk = jnp.zeros((RPT, H), jnp.float32)
    batch2 = batch.reshape(1, N)
    g2 = bn_gamma.reshape(1, H)
    bt2 = bn_beta.reshape(1, H)
    bl2 = b_lin.reshape(1, C)

    h = x
    layers = ((W_rel1, b_rel1, W_root1), (W_rel2, b_rel2, W_root2),
              (W_rel3, b_rel3, W_root3))
    for i, (Wr, br, Wt) in enumerate(layers):
        parts = _segment_sum_sc(h, e4, zblk)
        br2 = br.reshape(1, H)
        if i < 2:
            h = _dense_tc(parts, h, Wr, br2, Wt, g2, bt2)
        else:
            out = _dense_pool_tc(parts, h, Wr, br2, Wt, g2, bt2,
                                 batch2, W_lin, bl2)
    return out


# speedup vs baseline: 1.0008x; 1.0008x over previous
import functools

import jax
import jax.numpy as jnp
from jax import lax
from jax.experimental import pallas as pl
from jax.experimental.pallas import tpu as pltpu
from jax.experimental.pallas import tpu_sc as plsc

N = 10000
E = 320000
D = 128
H = 128
C = 32
G = 64

NC = 2
NS = 16
NW = NC * NS
BLK = 128
NBLK = 80
EPT = NBLK * BLK
EPAD = NW * EPT
NPAD = 10240
RPT = NPAD // NS

_mesh = plsc.VectorSubcoreMesh(core_axis_name="c", subcore_axis_name="s")


def _segment_sum_sc(h, e4, zblk):
    @functools.partial(
        pl.kernel,
        out_type=jax.ShapeDtypeStruct((NC * NPAD, H), jnp.float32),
        mesh=_mesh,
        scratch_types=[
            pltpu.VMEM((4, 2, BLK), jnp.int32),
            pltpu.VMEM((BLK, H), jnp.float32),
            pltpu.VMEM((BLK, H), jnp.float32),
            pltpu.SemaphoreType.DMA,
            pltpu.SemaphoreType.DMA,
            pltpu.SemaphoreType.DMA,
            pltpu.SemaphoreType.DMA,
            pltpu.SemaphoreType.DMA,
            pltpu.SemaphoreType.DMA,
            pltpu.VMEM_SHARED((NPAD, H), jnp.float32),
        ],
    )
    def k(h_hbm, e_hbm, z_hbm, out_hbm, ebuf, rows0, rows1,
          si0, si1, si2, si3, sg0, sg1, acc):
        c = lax.axis_index("c")
        s = lax.axis_index("s")
        tid = c * NS + s
        rows = (rows0, rows1)
        semi = (si0, si1, si2, si3)
        semg = (sg0, sg1)
        pltpu.sync_copy(z_hbm, acc.at[pl.ds(s * RPT, RPT)])
        for p in range(4):
            pltpu.async_copy(e_hbm.at[tid, p], ebuf.at[p], semi[p])
        plsc.subcore_barrier()
        pltpu.make_async_copy(e_hbm.at[tid, 0], ebuf.at[0], semi[0]).wait()
        pltpu.async_copy(h_hbm.at[ebuf.at[0, 0]], rows0, semg[0])

        def phase(j, p, pref_idx, pref_gather):
            b = p % 2
            if pref_gather:
                p1 = (p + 1) % 4
                pltpu.make_async_copy(e_hbm.at[tid, j + 1], ebuf.at[p1],
                                      semi[p1]).wait()
                pltpu.async_copy(h_hbm.at[ebuf.at[p1, 0]], rows[1 - b],
                                 semg[1 - b])
            pltpu.make_async_copy(h_hbm.at[ebuf.at[p, 0]], rows[b],
                                  semg[b]).wait()
            pltpu.sync_copy(rows[b], acc.at[ebuf.at[p, 1]], add=True)
            if pref_idx:
                pltpu.async_copy(e_hbm.at[tid, j + 4], ebuf.at[p], semi[p])

        @pl.loop(0, NBLK - 4, step=4)
        def _(j):
            for p in range(4):
                phase(j + p, p, True, True)

        for q in range(4):
            phase(NBLK - 4 + q, q, False, q < 3)

        plsc.subcore_barrier()
        pltpu.sync_copy(acc.at[pl.ds(s * RPT, RPT)],
                        out_hbm.at[pl.ds(c * NPAD + s * RPT, RPT)])

    return k(h, e4, zblk)


def _dense_body(p_ref, h_ref, wr_ref, br_ref, wt_ref, g_ref, b_ref, o_ref):
    agg = p_ref[0, :N] + p_ref[1, :N]
    y = (lax.dot(agg, wr_ref[...], precision=lax.Precision.HIGHEST,
                 preferred_element_type=jnp.float32)
         + lax.dot(h_ref[...], wt_ref[...], precision=lax.Precision.HIGHEST,
                   preferred_element_type=jnp.float32)
         + br_ref[...])
    y = jnp.maximum(y, 0.0)
    mean = jnp.mean(y, axis=0, keepdims=True)
    var = jnp.mean((y - mean) ** 2, axis=0, keepdims=True)
    o_ref[...] = (y - mean) * lax.rsqrt(var + 1e-5) * g_ref[...] + b_ref[...]


def _dense_tc(parts, h, wr, br, wt, gamma, beta):
    parts = parts.reshape(NC, NPAD, H)
    return pl.pallas_call(
        _dense_body,
        out_shape=jax.ShapeDtypeStruct((N, H), jnp.float32),
    )(parts, h, wr, br, wt, gamma, beta)


def _dense_pool_body(p_ref, h_ref, wr_ref, br_ref, wt_ref, g_ref, b_ref,
                     batch_ref, wl_ref, bl_ref, o_ref):
    agg = p_ref[0, :N] + p_ref[1, :N]
    y = (lax.dot(agg, wr_ref[...], precision=lax.Precision.HIGHEST,
                 preferred_element_type=jnp.float32)
         + lax.dot(h_ref[...], wt_ref[...], precision=lax.Precision.HIGHEST,
                   preferred_element_type=jnp.float32)
         + br_ref[...])
    y = jnp.maximum(y, 0.0)
    mean = jnp.mean(y, axis=0, keepdims=True)
    var = jnp.mean((y - mean) ** 2, axis=0, keepdims=True)
    y = (y - mean) * lax.rsqrt(var + 1e-5) * g_ref[...] + b_ref[...]
    onehot = (batch_ref[...] ==
              lax.broadcasted_iota(jnp.int32, (G, N), 0)).astype(jnp.float32)
    sums = lax.dot(onehot, y, precision=lax.Precision.HIGHEST,
                   preferred_element_type=jnp.float32)
    counts = jnp.sum(onehot, axis=1, keepdims=True)
    pooled = sums / jnp.maximum(counts, 1.0)
    o_ref[...] = (lax.dot(pooled, wl_ref[...], precision=lax.Precision.HIGHEST,
                          preferred_element_type=jnp.float32) + bl_ref[...])


def _dense_pool_tc(parts, h, wr, br, wt, gamma, beta, batch2, wl, bl):
    parts = parts.reshape(NC, NPAD, H)
    return pl.pallas_call(
        _dense_pool_body,
        out_shape=jax.ShapeDtypeStruct((G, C), jnp.float32),
    )(parts, h, wr, br, wt, gamma, beta, batch2, wl, bl)


def kernel(x, edge_index, batch, W_rel1, b_rel1, W_root1, W_rel2, b_rel2,
           W_root2, W_rel3, b_rel3, W_root3, bn_gamma, bn_beta, W_lin, b_lin):
    ppt = EPT - E // NW
    padsrc = jnp.broadcast_to((jnp.arange(ppt, dtype=jnp.int32) * 41) % N,
                              (NW, ppt))
    srcp = jnp.concatenate(
        [edge_index[0].reshape(NW, E // NW), padsrc], axis=1)
    drpt = (NPAD - N) // NS
    padrow = (N + (jnp.arange(NW, dtype=jnp.int32) % NS)[:, None] * drpt
              + (jnp.arange(ppt, dtype=jnp.int32) % drpt)[None, :])
    dstp = jnp.concatenate(
        [edge_index[1].reshape(NW, E // NW), padrow], axis=1)
    e4 = jnp.stack([srcp.reshape(NW, NBLK, BLK),
                    dstp.reshape(NW, NBLK, BLK)], axis=2)
    zblk = jnp.zeros((RPT, H), jnp.float32)
    batch2 = batch.reshape(1, N)
    g2 = bn_gamma.reshape(1, H)
    bt2 = bn_beta.reshape(1, H)
    bl2 = b_lin.reshape(1, C)

    h = x
    layers = ((W_rel1, b_rel1, W_root1), (W_rel2, b_rel2, W_root2),
              (W_rel3, b_rel3, W_root3))
    for i, (Wr, br, Wt) in enumerate(layers):
        parts = _segment_sum_sc(h, e4, zblk)
        br2 = br.reshape(1, H)
        if i < 2:
            h = _dense_tc(parts, h, Wr, br2, Wt, g2, bt2)
        else:
            out = _dense_pool_tc(parts, h, Wr, br2, Wt, g2, bt2,
                                 batch2, W_lin, bl2)
    return out


# speedup vs baseline: 1.0109x; 1.0100x over previous
import functools

import jax
import jax.numpy as jnp
from jax import lax
from jax.experimental import pallas as pl
from jax.experimental.pallas import tpu as pltpu
from jax.experimental.pallas import tpu_sc as plsc

N = 10000
E = 320000
D = 128
H = 128
C = 32
G = 64

NC = 2
NS = 16
NW = NC * NS
BLK = 128
NBLK = 80
EPT = NBLK * BLK
EPAD = NW * EPT
NPAD = 10240
RPT = NPAD // NS

_mesh = plsc.VectorSubcoreMesh(core_axis_name="c", subcore_axis_name="s")


def _segment_sum_sc(h, e4, zblk):
    @functools.partial(
        pl.kernel,
        out_type=jax.ShapeDtypeStruct((NC * NPAD, H), jnp.float32),
        mesh=_mesh,
        scratch_types=[
            pltpu.VMEM((4, 2, BLK), jnp.int32),
            pltpu.VMEM((BLK, H), jnp.float32),
            pltpu.VMEM((BLK, H), jnp.float32),
            pltpu.SemaphoreType.DMA,
            pltpu.SemaphoreType.DMA,
            pltpu.SemaphoreType.DMA,
            pltpu.SemaphoreType.DMA,
            pltpu.SemaphoreType.DMA,
            pltpu.SemaphoreType.DMA,
            pltpu.VMEM_SHARED((NPAD, H), jnp.float32),
        ],
    )
    def k(h_hbm, e_hbm, z_hbm, out_hbm, ebuf, rows0, rows1,
          si0, si1, si2, si3, sg0, sg1, acc):
        c = lax.axis_index("c")
        s = lax.axis_index("s")
        tid = c * NS + s
        rows = (rows0, rows1)
        semi = (si0, si1, si2, si3)
        semg = (sg0, sg1)
        for p in range(4):
            pltpu.async_copy(e_hbm.at[tid, p], ebuf.at[p], semi[p])
        pltpu.sync_copy(z_hbm, acc.at[pl.ds(s * RPT, RPT)])
        pltpu.make_async_copy(e_hbm.at[tid, 0], ebuf.at[0], semi[0]).wait()
        pltpu.async_copy(h_hbm.at[ebuf.at[0, 0]], rows0, semg[0])
        plsc.subcore_barrier()

        def phase(j, p, pref_idx, pref_gather):
            b = p % 2
            if pref_gather:
                p1 = (p + 1) % 4
                pltpu.make_async_copy(e_hbm.at[tid, j + 1], ebuf.at[p1],
                                      semi[p1]).wait()
                pltpu.async_copy(h_hbm.at[ebuf.at[p1, 0]], rows[1 - b],
                                 semg[1 - b])
            pltpu.make_async_copy(h_hbm.at[ebuf.at[p, 0]], rows[b],
                                  semg[b]).wait()
            pltpu.sync_copy(rows[b], acc.at[ebuf.at[p, 1]], add=True)
            if pref_idx:
                pltpu.async_copy(e_hbm.at[tid, j + 4], ebuf.at[p], semi[p])

        @pl.loop(0, NBLK - 4, step=4)
        def _(j):
            for p in range(4):
                phase(j + p, p, True, True)

        for q in range(4):
            phase(NBLK - 4 + q, q, False, q < 3)

        plsc.subcore_barrier()
        pltpu.sync_copy(acc.at[pl.ds(s * RPT, RPT)],
                        out_hbm.at[pl.ds(c * NPAD + s * RPT, RPT)])

    return k(h, e4, zblk)


def _dense_body(p_ref, h_ref, wr_ref, br_ref, wt_ref, g_ref, b_ref, o_ref):
    agg = p_ref[0, :N] + p_ref[1, :N]
    y = (lax.dot(agg, wr_ref[...], precision=lax.Precision.HIGHEST,
                 preferred_element_type=jnp.float32)
         + lax.dot(h_ref[...], wt_ref[...], precision=lax.Precision.HIGHEST,
                   preferred_element_type=jnp.float32)
         + br_ref[...])
    y = jnp.maximum(y, 0.0)
    mean = jnp.mean(y, axis=0, keepdims=True)
    var = jnp.mean((y - mean) ** 2, axis=0, keepdims=True)
    o_ref[...] = (y - mean) * lax.rsqrt(var + 1e-5) * g_ref[...] + b_ref[...]


def _dense_tc(parts, h, wr, br, wt, gamma, beta):
    parts = parts.reshape(NC, NPAD, H)
    return pl.pallas_call(
        _dense_body,
        out_shape=jax.ShapeDtypeStruct((N, H), jnp.float32),
    )(parts, h, wr, br, wt, gamma, beta)


def _dense_pool_body(p_ref, h_ref, wr_ref, br_ref, wt_ref, g_ref, b_ref,
                     batch_ref, wl_ref, bl_ref, o_ref):
    agg = p_ref[0, :N] + p_ref[1, :N]
    y = (lax.dot(agg, wr_ref[...], precision=lax.Precision.HIGHEST,
                 preferred_element_type=jnp.float32)
         + lax.dot(h_ref[...], wt_ref[...], precision=lax.Precision.HIGHEST,
                   preferred_element_type=jnp.float32)
         + br_ref[...])
    y = jnp.maximum(y, 0.0)
    mean = jnp.mean(y, axis=0, keepdims=True)
    var = jnp.mean((y - mean) ** 2, axis=0, keepdims=True)
    y = (y - mean) * lax.rsqrt(var + 1e-5) * g_ref[...] + b_ref[...]
    onehot = (batch_ref[...] ==
              lax.broadcasted_iota(jnp.int32, (G, N), 0)).astype(jnp.float32)
    sums = lax.dot(onehot, y, precision=lax.Precision.HIGHEST,
                   preferred_element_type=jnp.float32)
    counts = jnp.sum(onehot, axis=1, keepdims=True)
    pooled = sums / jnp.maximum(counts, 1.0)
    o_ref[...] = (lax.dot(pooled, wl_ref[...], precision=lax.Precision.HIGHEST,
                          preferred_element_type=jnp.float32) + bl_ref[...])


def _dense_pool_tc(parts, h, wr, br, wt, gamma, beta, batch2, wl, bl):
    parts = parts.reshape(NC, NPAD, H)
    return pl.pallas_call(
        _dense_pool_body,
        out_shape=jax.ShapeDtypeStruct((G, C), jnp.float32),
    )(parts, h, wr, br, wt, gamma, beta, batch2, wl, bl)


def kernel(x, edge_index, batch, W_rel1, b_rel1, W_root1, W_rel2, b_rel2,
           W_root2, W_rel3, b_rel3, W_root3, bn_gamma, bn_beta, W_lin, b_lin):
    ppt = EPT - E // NW
    padsrc = jnp.broadcast_to((jnp.arange(ppt, dtype=jnp.int32) * 41) % N,
                              (NW, ppt))
    srcp = jnp.concatenate(
        [edge_index[0].reshape(NW, E // NW), padsrc], axis=1)
    drpt = (NPAD - N) // NS
    padrow = (N + (jnp.arange(NW, dtype=jnp.int32) % NS)[:, None] * drpt
              + (jnp.arange(ppt, dtype=jnp.int32) % drpt)[None, :])
    dstp = jnp.concatenate(
        [edge_index[1].reshape(NW, E // NW), padrow], axis=1)
    e4 = jnp.stack([srcp.reshape(NW, NBLK, BLK),
                    dstp.reshape(NW, NBLK, BLK)], axis=2)
    zblk = jnp.zeros((RPT, H), jnp.float32)
    batch2 = batch.reshape(1, N)
    g2 = bn_gamma.reshape(1, H)
    bt2 = bn_beta.reshape(1, H)
    bl2 = b_lin.reshape(1, C)

    h = x
    layers = ((W_rel1, b_rel1, W_root1), (W_rel2, b_rel2, W_root2),
              (W_rel3, b_rel3, W_root3))
    for i, (Wr, br, Wt) in enumerate(layers):
        parts = _segment_sum_sc(h, e4, zblk)
        br2 = br.reshape(1, H)
        if i < 2:
            h = _dense_tc(parts, h, Wr, br2, Wt, g2, bt2)
        else:
            out = _dense_pool_tc(parts, h, Wr, br2, Wt, g2, bt2,
                                 batch2, W_lin, bl2)
    return out
